# baseline (speedup 1.0000x reference)
import functools

import jax
import jax.numpy as jnp
from jax.experimental import pallas as pl
from jax.experimental.pallas import tpu as pltpu
from jax.experimental.pallas import tpu_sc as plsc

_T = 256
_TR = 512


def _gelu_exact(x):
    return 0.5 * x * (1.0 + jax.lax.erf(x * (2.0 ** -0.5)))


def _cumsum0(a):
    n = a.shape[0]
    s = 1
    while s < n:
        a = a + jnp.concatenate(
            [jnp.zeros((s, a.shape[1]), a.dtype), a[:n - s]], axis=0)
        s *= 2
    return a


def _cumsum1(a):
    n = a.shape[1]
    s = 1
    while s < n:
        a = a + jnp.concatenate(
            [jnp.zeros((a.shape[0], s), a.dtype), a[:, :n - s]], axis=1)
        s *= 2
    return a


def _dispatch_body(x_ref, gw_ref, gb_ref,
                   pos_ref, ts_ref, te1_ref, lo_ref, hi_ref):
    N = x_ref.shape[0]
    E = gw_ref.shape[0]
    T = _T

    logits = jax.lax.dot_general(
        x_ref[...], gw_ref[...], (((1,), (1,)), ((), ())),
        preferred_element_type=jnp.float32,
        precision=jax.lax.Precision.DEFAULT,
    ) + gb_ref[...]

    lane = jax.lax.broadcasted_iota(jnp.int32, (N, E), 1)
    maxv = jnp.max(logits, axis=1, keepdims=True)
    eids = jnp.min(jnp.where(logits == maxv, lane, E), axis=1, keepdims=True)

    onehot = (eids == lane).astype(jnp.int32)
    counts = jnp.sum(onehot, axis=0, keepdims=True)
    cend = _cumsum1(counts)
    start = cend - counts
    rank = _cumsum0(onehot) - onehot
    pos = jnp.sum(onehot * (start + rank), axis=1, keepdims=True)
    pos_ref[...] = pos.astype(jnp.int32)

    ts = start // T
    te = (cend - 1) // T
    empty = counts == 0
    ts_ref[...] = jnp.where(empty, 0, ts).astype(jnp.int32).reshape(E, 1)
    te1_ref[...] = jnp.where(empty, 0, te + 1).astype(jnp.int32).reshape(E, 1)
    lo_ref[...] = start.astype(jnp.int32).reshape(E, 1)
    hi_ref[...] = cend.astype(jnp.int32).reshape(E, 1)


_SC_W = 128


def _sc_permute_rows(src, idx2, direction):
    N, D = src.shape
    W = _SC_W
    NW = N // W
    mesh = plsc.VectorSubcoreMesh(core_axis_name="core",
                                  subcore_axis_name="subcore")
    nu = mesh.num_cores * mesh.num_subcores
    jmax = (NW + nu - 1) // nu

    H = W // 2

    @pl.kernel(out_type=jax.ShapeDtypeStruct((N, D), src.dtype), mesh=mesh,
               scratch_types=[pltpu.VMEM((1, W), jnp.int32),
                              pltpu.VMEM((H, D), src.dtype),
                              pltpu.SemaphoreType.DMA])
    def k(s_hbm, i_hbm, o_hbm, ibuf, dbuf, sem):
        c = jax.lax.axis_index("core")
        s = jax.lax.axis_index("subcore")
        u = c * mesh.num_subcores + s

        @pl.loop(0, jmax)
        def _(j):
            w = u + j * nu

            @pl.when(w < NW)
            def _():
                pltpu.async_copy(
                    i_hbm.at[:, pl.ds(w * W, W)], ibuf, sem).wait()
                for h in range(W // H):
                    iview = ibuf.at[0, pl.ds(h * H, H)]
                    rows = pl.ds(w * W + h * H, H)
                    if direction == "gather":
                        pltpu.async_copy(s_hbm.at[iview], dbuf, sem).wait()
                        pltpu.async_copy(dbuf, o_hbm.at[rows], sem).wait()
                    else:
                        pltpu.async_copy(s_hbm.at[rows], dbuf, sem).wait()
                        pltpu.async_copy(dbuf, o_hbm.at[iview], sem).wait()

    return k(src, idx2)


def _sc_scatter_rows(x, idx2):
    return _sc_permute_rows(x, idx2, "scatter")


def _sc_gather_rows(src, idx2):
    return _sc_permute_rows(src, idx2, "gather")


_DEPTH = 3


def _ffn_body(ts_ref, te1_ref, lo_ref, hi_ref,
              x_any, w1_any, b1_ref, w2_any, b2_ref, o_ref,
              x_ref, w1buf, w2buf, semx, sem1, sem2):
    E = b1_ref.shape[0]
    Fh = w1buf.shape[1]
    T = _T
    S = 2 * E

    def issue(s):
        e = s // 2
        fh = s % 2
        slot = s % _DEPTH
        pltpu.make_async_copy(
            w1_any.at[e, pl.ds(fh * Fh, Fh), :], w1buf.at[slot],
            sem1.at[slot]).start()
        pltpu.make_async_copy(
            w2_any.at[e, :, pl.ds(fh * Fh, Fh)], w2buf.at[slot],
            sem2.at[slot]).start()

    xcopy = pltpu.make_async_copy(x_any, x_ref, semx)
    xcopy.start()
    for s in range(_DEPTH):
        issue(s)
    xcopy.wait()

    def stage(s, carry):
        e = s // 2
        fh = s % 2
        slot = s % _DEPTH
        pltpu.make_async_copy(
            w1_any.at[e, pl.ds(fh * Fh, Fh), :], w1buf.at[slot],
            sem1.at[slot]).wait()
        pltpu.make_async_copy(
            w2_any.at[e, :, pl.ds(fh * Fh, Fh)], w2buf.at[slot],
            sem2.at[slot]).wait()

        lo = lo_ref[e]
        hi = hi_ref[e]
        w1 = w1buf[slot].astype(jnp.bfloat16)
        w2 = w2buf[slot].astype(jnp.bfloat16)
        b1h = b1_ref[e, 0, pl.ds(fh * Fh, Fh)]
        b2v = b2_ref[e]

        def tile_body(t, c):
            xb = x_ref[pl.ds(t * T, T), :].astype(jnp.bfloat16)
            h = jax.lax.dot_general(
                xb, w1, (((1,), (1,)), ((), ())),
                preferred_element_type=jnp.float32)
            h = _gelu_exact(h + b1h)
            y = jax.lax.dot_general(
                h.astype(jnp.bfloat16), w2, (((1,), (1,)), ((), ())),
                preferred_element_type=jnp.float32)
            y = y + b2v * jnp.where(fh == 0, 1.0, 0.0)
            rowid = jax.lax.broadcasted_iota(jnp.int32, (T, 1), 0) + t * T
            mask = (rowid >= lo) & (rowid < hi)
            first_owner = (fh == 0) & (t * T >= lo)

            @pl.when(first_owner)
            def _init():
                o_ref[pl.ds(t * T, T), :] = jnp.where(mask, y, 0.0)

            @pl.when(jnp.logical_not(first_owner))
            def _accum():
                o_ref[pl.ds(t * T, T), :] += jnp.where(mask, y, 0.0)

            return c

        jax.lax.fori_loop(ts_ref[e], te1_ref[e], tile_body, 0)

        @pl.when(s + _DEPTH < S)
        def _prefetch():
            issue_dyn(s + _DEPTH)

        return carry

    def issue_dyn(s):
        e = s // 2
        fh = s % 2
        slot = s % _DEPTH
        pltpu.make_async_copy(
            w1_any.at[e, pl.ds(fh * Fh, Fh), :], w1buf.at[slot],
            sem1.at[slot]).start()
        pltpu.make_async_copy(
            w2_any.at[e, :, pl.ds(fh * Fh, Fh)], w2buf.at[slot],
            sem2.at[slot]).start()

    jax.lax.fori_loop(0, S, stage, 0)


def kernel(x, gate_W, gate_b, W1, b1, W2, b2):
    Bd, L, Dd = x.shape
    E, F, _ = W1.shape
    N = Bd * L
    T = _T
    NT = N // T
    MAXP = (NT + E - 1 + 7) // 8 * 8

    x_flat = x.reshape(N, Dd)

    i32 = jnp.int32
    pos2, ts_a, te1_a, lo_a, hi_a = pl.pallas_call(
        _dispatch_body,
        grid=(1,),
        in_specs=[
            pl.BlockSpec((N, Dd), lambda i: (0, 0)),
            pl.BlockSpec((E, Dd), lambda i: (0, 0)),
            pl.BlockSpec((1, E), lambda i: (0, 0)),
        ],
        out_specs=[
            pl.BlockSpec((N, 1), lambda i: (0, 0)),
            pl.BlockSpec((E, 1), lambda i: (0, 0)),
            pl.BlockSpec((E, 1), lambda i: (0, 0)),
            pl.BlockSpec((E, 1), lambda i: (0, 0)),
            pl.BlockSpec((E, 1), lambda i: (0, 0)),
        ],
        out_shape=[
            jax.ShapeDtypeStruct((N, 1), i32),
            jax.ShapeDtypeStruct((E, 1), i32),
            jax.ShapeDtypeStruct((E, 1), i32),
            jax.ShapeDtypeStruct((E, 1), i32),
            jax.ShapeDtypeStruct((E, 1), i32),
        ],
    )(x_flat, gate_W, gate_b.reshape(1, E))
    posr = pos2.reshape(1, N)
    ts_a, te1_a, lo_a, hi_a = (
        a.reshape(E) for a in (ts_a, te1_a, lo_a, hi_a))

    x_sorted = _sc_scatter_rows(x_flat, posr)

    Fh = F // 2
    f32 = jnp.float32
    grid_spec = pltpu.PrefetchScalarGridSpec(
        num_scalar_prefetch=4,
        grid=(1,),
        in_specs=[
            pl.BlockSpec(memory_space=pltpu.MemorySpace.HBM),
            pl.BlockSpec(memory_space=pltpu.MemorySpace.HBM),
            pl.BlockSpec((E, 1, F), lambda i, *s: (0, 0, 0)),
            pl.BlockSpec(memory_space=pltpu.MemorySpace.HBM),
            pl.BlockSpec((E, 1, Dd), lambda i, *s: (0, 0, 0)),
        ],
        out_specs=pl.BlockSpec((N, Dd), lambda i, *s: (0, 0)),
        scratch_shapes=[
            pltpu.VMEM((N, Dd), f32),
            pltpu.VMEM((_DEPTH, Fh, Dd), f32),
            pltpu.VMEM((_DEPTH, Dd, Fh), f32),
            pltpu.SemaphoreType.DMA,
            pltpu.SemaphoreType.DMA((_DEPTH,)),
            pltpu.SemaphoreType.DMA((_DEPTH,)),
        ],
    )
    out_sorted = pl.pallas_call(
        _ffn_body,
        grid_spec=grid_spec,
        out_shape=jax.ShapeDtypeStruct((N, Dd), jnp.float32),
        compiler_params=pltpu.CompilerParams(vmem_limit_bytes=63 << 20),
    )(ts_a, te1_a, lo_a, hi_a,
      x_sorted, W1, b1.reshape(E, 1, F), W2, b2.reshape(E, 1, Dd))

    out = _sc_gather_rows(out_sorted, posr)
    return out.reshape(Bd, L, Dd)


# speedup vs baseline: 1.0158x; 1.0158x over previous
import jax
import jax.numpy as jnp
from jax.experimental import pallas as pl
from jax.experimental.pallas import tpu as pltpu
from jax.experimental.pallas import tpu_sc as plsc

_T = 256


def _gelu_exact(x):
    return 0.5 * x * (1.0 + jax.lax.erf(x * (2.0 ** -0.5)))


def _cumsum0(a):
    n = a.shape[0]
    s = 1
    while s < n:
        a = a + jnp.concatenate(
            [jnp.zeros((s, a.shape[1]), a.dtype), a[:n - s]], axis=0)
        s *= 2
    return a


def _cumsum1(a):
    n = a.shape[1]
    s = 1
    while s < n:
        a = a + jnp.concatenate(
            [jnp.zeros((a.shape[0], s), a.dtype), a[:, :n - s]], axis=1)
        s *= 2
    return a


def _dispatch_body(x_ref, gw_ref, gb_ref,
                   pos_ref, ts_ref, te1_ref, lo_ref, hi_ref):
    N = x_ref.shape[0]
    E = gw_ref.shape[0]
    T = _T

    logits = jax.lax.dot_general(
        x_ref[...], gw_ref[...], (((1,), (1,)), ((), ())),
        preferred_element_type=jnp.float32,
        precision=jax.lax.Precision.DEFAULT,
    ) + gb_ref[...]

    lane = jax.lax.broadcasted_iota(jnp.int32, (N, E), 1)
    maxv = jnp.max(logits, axis=1, keepdims=True)
    eids = jnp.min(jnp.where(logits == maxv, lane, E), axis=1, keepdims=True)

    onehot = (eids == lane).astype(jnp.int32)
    counts = jnp.sum(onehot, axis=0, keepdims=True)
    cend = _cumsum1(counts)
    start = cend - counts
    rank = _cumsum0(onehot) - onehot
    pos = jnp.sum(onehot * (start + rank), axis=1, keepdims=True)
    pos_ref[...] = pos.astype(jnp.int32)

    ts = start // T
    te = (cend - 1) // T
    empty = counts == 0
    ts_ref[...] = jnp.where(empty, 0, ts).astype(jnp.int32).reshape(E, 1)
    te1_ref[...] = jnp.where(empty, 0, te + 1).astype(jnp.int32).reshape(E, 1)
    lo_ref[...] = start.astype(jnp.int32).reshape(E, 1)
    hi_ref[...] = cend.astype(jnp.int32).reshape(E, 1)


_SC_W = 128


def _sc_permute_rows(src, idx2, direction):
    N, D = src.shape
    W = _SC_W
    NW = N // W
    mesh = plsc.VectorSubcoreMesh(core_axis_name="core",
                                  subcore_axis_name="subcore")
    nu = mesh.num_cores * mesh.num_subcores
    jmax = (NW + nu - 1) // nu

    H = W // 2

    @pl.kernel(out_type=jax.ShapeDtypeStruct((N, D), src.dtype), mesh=mesh,
               scratch_types=[pltpu.VMEM((1, W), jnp.int32),
                              pltpu.VMEM((H, D), src.dtype),
                              pltpu.SemaphoreType.DMA])
    def k(s_hbm, i_hbm, o_hbm, ibuf, dbuf, sem):
        c = jax.lax.axis_index("core")
        s = jax.lax.axis_index("subcore")
        u = c * mesh.num_subcores + s

        @pl.loop(0, jmax)
        def _(j):
            w = u + j * nu

            @pl.when(w < NW)
            def _():
                pltpu.async_copy(
                    i_hbm.at[:, pl.ds(w * W, W)], ibuf, sem).wait()
                for h in range(W // H):
                    iview = ibuf.at[0, pl.ds(h * H, H)]
                    rows = pl.ds(w * W + h * H, H)
                    if direction == "gather":
                        pltpu.async_copy(s_hbm.at[iview], dbuf, sem).wait()
                        pltpu.async_copy(dbuf, o_hbm.at[rows], sem).wait()
                    else:
                        pltpu.async_copy(s_hbm.at[rows], dbuf, sem).wait()
                        pltpu.async_copy(dbuf, o_hbm.at[iview], sem).wait()

    return k(src, idx2)


def _sc_scatter_rows(x, idx2):
    return _sc_permute_rows(x, idx2, "scatter")


def _sc_gather_rows(src, idx2):
    return _sc_permute_rows(src, idx2, "gather")


_DEPTH = 3


def _ffn_body(ts_ref, te1_ref, lo_ref, hi_ref,
              x_ref, w1_any, b1_ref, w2_any, b2_ref, o_ref,
              w1buf, w2buf, sem1, sem2):
    E = b1_ref.shape[0]
    Fh = w1buf.shape[1]
    T = _T
    S = 2 * E

    def issue(s):
        e = s // 2
        fh = s % 2
        slot = s % _DEPTH
        pltpu.make_async_copy(
            w1_any.at[e, pl.ds(fh * Fh, Fh), :], w1buf.at[slot],
            sem1.at[slot]).start()
        pltpu.make_async_copy(
            w2_any.at[e, :, pl.ds(fh * Fh, Fh)], w2buf.at[slot],
            sem2.at[slot]).start()

    for s in range(_DEPTH):
        issue(s)

    def stage(s, carry):
        e = s // 2
        fh = s % 2
        slot = s % _DEPTH
        pltpu.make_async_copy(
            w1_any.at[e, pl.ds(fh * Fh, Fh), :], w1buf.at[slot],
            sem1.at[slot]).wait()
        pltpu.make_async_copy(
            w2_any.at[e, :, pl.ds(fh * Fh, Fh)], w2buf.at[slot],
            sem2.at[slot]).wait()

        lo = lo_ref[e]
        hi = hi_ref[e]
        w1 = w1buf[slot].astype(jnp.bfloat16)
        w2 = w2buf[slot].astype(jnp.bfloat16)
        b1h = b1_ref[e, 0, pl.ds(fh * Fh, Fh)]
        b2v = b2_ref[e]

        def tile_body(t, c):
            xb = x_ref[pl.ds(t * T, T), :].astype(jnp.bfloat16)
            h = jax.lax.dot_general(
                xb, w1, (((1,), (1,)), ((), ())),
                preferred_element_type=jnp.float32)
            h = _gelu_exact(h + b1h)
            y = jax.lax.dot_general(
                h.astype(jnp.bfloat16), w2, (((1,), (1,)), ((), ())),
                preferred_element_type=jnp.float32)
            y = y + b2v * jnp.where(fh == 0, 1.0, 0.0)
            rowid = jax.lax.broadcasted_iota(jnp.int32, (T, 1), 0) + t * T
            mask = (rowid >= lo) & (rowid < hi)
            first_owner = (fh == 0) & (t * T >= lo)

            @pl.when(first_owner)
            def _init():
                o_ref[pl.ds(t * T, T), :] = jnp.where(mask, y, 0.0)

            @pl.when(jnp.logical_not(first_owner))
            def _accum():
                o_ref[pl.ds(t * T, T), :] += jnp.where(mask, y, 0.0)

            return c

        jax.lax.fori_loop(ts_ref[e], te1_ref[e], tile_body, 0)

        @pl.when(s + _DEPTH < S)
        def _prefetch():
            issue_dyn(s + _DEPTH)

        return carry

    def issue_dyn(s):
        e = s // 2
        fh = s % 2
        slot = s % _DEPTH
        pltpu.make_async_copy(
            w1_any.at[e, pl.ds(fh * Fh, Fh), :], w1buf.at[slot],
            sem1.at[slot]).start()
        pltpu.make_async_copy(
            w2_any.at[e, :, pl.ds(fh * Fh, Fh)], w2buf.at[slot],
            sem2.at[slot]).start()

    jax.lax.fori_loop(0, S, stage, 0)


def kernel(x, gate_W, gate_b, W1, b1, W2, b2):
    Bd, L, Dd = x.shape
    E, F, _ = W1.shape
    N = Bd * L
    T = _T

    x_flat = x.reshape(N, Dd)

    i32 = jnp.int32
    pos2, ts_a, te1_a, lo_a, hi_a = pl.pallas_call(
        _dispatch_body,
        grid=(1,),
        in_specs=[
            pl.BlockSpec((N, Dd), lambda i: (0, 0)),
            pl.BlockSpec((E, Dd), lambda i: (0, 0)),
            pl.BlockSpec((1, E), lambda i: (0, 0)),
        ],
        out_specs=[
            pl.BlockSpec((N, 1), lambda i: (0, 0)),
            pl.BlockSpec((E, 1), lambda i: (0, 0)),
            pl.BlockSpec((E, 1), lambda i: (0, 0)),
            pl.BlockSpec((E, 1), lambda i: (0, 0)),
            pl.BlockSpec((E, 1), lambda i: (0, 0)),
        ],
        out_shape=[
            jax.ShapeDtypeStruct((N, 1), i32),
            jax.ShapeDtypeStruct((E, 1), i32),
            jax.ShapeDtypeStruct((E, 1), i32),
            jax.ShapeDtypeStruct((E, 1), i32),
            jax.ShapeDtypeStruct((E, 1), i32),
        ],
    )(x_flat, gate_W, gate_b.reshape(1, E))
    posr = pos2.reshape(1, N)
    ts_a, te1_a, lo_a, hi_a = (
        a.reshape(E) for a in (ts_a, te1_a, lo_a, hi_a))

    x_sorted = _sc_scatter_rows(x_flat, posr)

    Fh = F // 2
    f32 = jnp.float32
    grid_spec = pltpu.PrefetchScalarGridSpec(
        num_scalar_prefetch=4,
        grid=(1,),
        in_specs=[
            pl.BlockSpec((N, Dd), lambda i, *s: (0, 0)),
            pl.BlockSpec(memory_space=pltpu.MemorySpace.HBM),
            pl.BlockSpec((E, 1, F), lambda i, *s: (0, 0, 0)),
            pl.BlockSpec(memory_space=pltpu.MemorySpace.HBM),
            pl.BlockSpec((E, 1, Dd), lambda i, *s: (0, 0, 0)),
        ],
        out_specs=pl.BlockSpec((N, Dd), lambda i, *s: (0, 0)),
        scratch_shapes=[
            pltpu.VMEM((_DEPTH, Fh, Dd), f32),
            pltpu.VMEM((_DEPTH, Dd, Fh), f32),
            pltpu.SemaphoreType.DMA((_DEPTH,)),
            pltpu.SemaphoreType.DMA((_DEPTH,)),
        ],
    )
    out_sorted = pl.pallas_call(
        _ffn_body,
        grid_spec=grid_spec,
        out_shape=jax.ShapeDtypeStruct((N, Dd), jnp.float32),
        compiler_params=pltpu.CompilerParams(vmem_limit_bytes=63 << 20),
    )(ts_a, te1_a, lo_a, hi_a,
      x_sorted, W1, b1.reshape(E, 1, F), W2, b2.reshape(E, 1, Dd))

    out = _sc_gather_rows(out_sorted, posr)
    return out.reshape(Bd, L, Dd)


# speedup vs baseline: 1.0225x; 1.0067x over previous
import jax
import jax.numpy as jnp
from jax.experimental import pallas as pl
from jax.experimental.pallas import tpu as pltpu
from jax.experimental.pallas import tpu_sc as plsc

_T = 256


def _gelu_exact(x):
    return 0.5 * x * (1.0 + jax.lax.erf(x * (2.0 ** -0.5)))


def _cumsum0(a):
    n = a.shape[0]
    s = 1
    while s < n:
        a = a + jnp.concatenate(
            [jnp.zeros((s, a.shape[1]), a.dtype), a[:n - s]], axis=0)
        s *= 2
    return a


def _cumsum1(a):
    n = a.shape[1]
    s = 1
    while s < n:
        a = a + jnp.concatenate(
            [jnp.zeros((a.shape[0], s), a.dtype), a[:, :n - s]], axis=1)
        s *= 2
    return a


def _dispatch_body(x_ref, gw_ref, gb_ref,
                   pos_ref, ts_ref, te1_ref, lo_ref, hi_ref):
    N = x_ref.shape[0]
    E = gw_ref.shape[0]
    T = _T

    logits = jax.lax.dot_general(
        x_ref[...], gw_ref[...], (((1,), (1,)), ((), ())),
        preferred_element_type=jnp.float32,
        precision=jax.lax.Precision.DEFAULT,
    ) + gb_ref[...]

    lane = jax.lax.broadcasted_iota(jnp.int32, (N, E), 1)
    maxv = jnp.max(logits, axis=1, keepdims=True)
    eids = jnp.min(jnp.where(logits == maxv, lane, E), axis=1, keepdims=True)

    onehot = (eids == lane).astype(jnp.int32)
    counts = jnp.sum(onehot, axis=0, keepdims=True)
    cend = _cumsum1(counts)
    start = cend - counts
    rank = _cumsum0(onehot) - onehot
    pos = jnp.sum(onehot * (start + rank), axis=1, keepdims=True)
    pos_ref[...] = pos.astype(jnp.int32)

    ts = start // T
    te = (cend - 1) // T
    empty = counts == 0
    ts_ref[...] = jnp.where(empty, 0, ts).astype(jnp.int32).reshape(E, 1)
    te1_ref[...] = jnp.where(empty, 0, te + 1).astype(jnp.int32).reshape(E, 1)
    lo_ref[...] = start.astype(jnp.int32).reshape(E, 1)
    hi_ref[...] = cend.astype(jnp.int32).reshape(E, 1)


_SC_W = 128


def _sc_permute_rows(src, idx2, direction):
    N, D = src.shape
    W = _SC_W
    NW = N // W
    mesh = plsc.VectorSubcoreMesh(core_axis_name="core",
                                  subcore_axis_name="subcore")
    nu = mesh.num_cores * mesh.num_subcores
    jmax = (NW + nu - 1) // nu

    H = W // 2

    @pl.kernel(out_type=jax.ShapeDtypeStruct((N, D), src.dtype), mesh=mesh,
               scratch_types=[pltpu.VMEM((1, W), jnp.int32),
                              pltpu.VMEM((H, D), src.dtype),
                              pltpu.VMEM((H, D), src.dtype),
                              pltpu.SemaphoreType.DMA,
                              pltpu.SemaphoreType.DMA])
    def k(s_hbm, i_hbm, o_hbm, ibuf, dbuf0, dbuf1, sem0, sem1):
        c = jax.lax.axis_index("core")
        s = jax.lax.axis_index("subcore")
        u = c * mesh.num_subcores + s

        @pl.loop(0, jmax)
        def _(j):
            w = u + j * nu

            @pl.when(w < NW)
            def _():
                pltpu.async_copy(
                    i_hbm.at[:, pl.ds(w * W, W)], ibuf, sem0).wait()
                halves = []
                for h, (dbuf, sem) in enumerate(((dbuf0, sem0),
                                                 (dbuf1, sem1))):
                    iview = ibuf.at[0, pl.ds(h * H, H)]
                    rows = pl.ds(w * W + h * H, H)
                    src_ref = s_hbm.at[iview] if direction == "gather" \
                        else s_hbm.at[rows]
                    dst_ref = o_hbm.at[rows] if direction == "gather" \
                        else o_hbm.at[iview]
                    cp_in = pltpu.async_copy(src_ref, dbuf, sem)
                    halves.append((cp_in, dbuf, dst_ref, sem))
                for cp_in, dbuf, dst_ref, sem in halves:
                    cp_in.wait()
                    pltpu.async_copy(dbuf, dst_ref, sem).wait()

    return k(src, idx2)


def _sc_scatter_rows(x, idx2):
    return _sc_permute_rows(x, idx2, "scatter")


def _sc_gather_rows(src, idx2):
    return _sc_permute_rows(src, idx2, "gather")


_DEPTH = 3


def _ffn_body(ts_ref, te1_ref, lo_ref, hi_ref,
              x_ref, w1_any, b1_ref, w2_any, b2_ref, o_ref,
              w1buf, w2buf, sem1, sem2):
    E = b1_ref.shape[0]
    Fh = w1buf.shape[1]
    T = _T
    S = 2 * E

    def issue(s):
        e = s // 2
        fh = s % 2
        slot = s % _DEPTH
        pltpu.make_async_copy(
            w1_any.at[e, pl.ds(fh * Fh, Fh), :], w1buf.at[slot],
            sem1.at[slot]).start()
        pltpu.make_async_copy(
            w2_any.at[e, :, pl.ds(fh * Fh, Fh)], w2buf.at[slot],
            sem2.at[slot]).start()

    for s in range(_DEPTH):
        issue(s)

    def stage(s, carry):
        e = s // 2
        fh = s % 2
        slot = s % _DEPTH
        pltpu.make_async_copy(
            w1_any.at[e, pl.ds(fh * Fh, Fh), :], w1buf.at[slot],
            sem1.at[slot]).wait()
        pltpu.make_async_copy(
            w2_any.at[e, :, pl.ds(fh * Fh, Fh)], w2buf.at[slot],
            sem2.at[slot]).wait()

        lo = lo_ref[e]
        hi = hi_ref[e]
        w1 = w1buf[slot].astype(jnp.bfloat16)
        w2 = w2buf[slot].astype(jnp.bfloat16)
        b1h = b1_ref[e, 0, pl.ds(fh * Fh, Fh)]
        b2v = b2_ref[e]

        def tile_body(t, c):
            xb = x_ref[pl.ds(t * T, T), :].astype(jnp.bfloat16)
            h = jax.lax.dot_general(
                xb, w1, (((1,), (1,)), ((), ())),
                preferred_element_type=jnp.float32)
            h = _gelu_exact(h + b1h)
            y = jax.lax.dot_general(
                h.astype(jnp.bfloat16), w2, (((1,), (1,)), ((), ())),
                preferred_element_type=jnp.float32)
            y = y + b2v * jnp.where(fh == 0, 1.0, 0.0)
            rowid = jax.lax.broadcasted_iota(jnp.int32, (T, 1), 0) + t * T
            mask = (rowid >= lo) & (rowid < hi)
            first_owner = (fh == 0) & (t * T >= lo)

            @pl.when(first_owner)
            def _init():
                o_ref[pl.ds(t * T, T), :] = jnp.where(mask, y, 0.0)

            @pl.when(jnp.logical_not(first_owner))
            def _accum():
                o_ref[pl.ds(t * T, T), :] += jnp.where(mask, y, 0.0)

            return c

        jax.lax.fori_loop(ts_ref[e], te1_ref[e], tile_body, 0)

        @pl.when(s + _DEPTH < S)
        def _prefetch():
            issue_dyn(s + _DEPTH)

        return carry

    def issue_dyn(s):
        e = s // 2
        fh = s % 2
        slot = s % _DEPTH
        pltpu.make_async_copy(
            w1_any.at[e, pl.ds(fh * Fh, Fh), :], w1buf.at[slot],
            sem1.at[slot]).start()
        pltpu.make_async_copy(
            w2_any.at[e, :, pl.ds(fh * Fh, Fh)], w2buf.at[slot],
            sem2.at[slot]).start()

    jax.lax.fori_loop(0, S, stage, 0)


def kernel(x, gate_W, gate_b, W1, b1, W2, b2):
    Bd, L, Dd = x.shape
    E, F, _ = W1.shape
    N = Bd * L
    T = _T

    x_flat = x.reshape(N, Dd)

    i32 = jnp.int32
    pos2, ts_a, te1_a, lo_a, hi_a = pl.pallas_call(
        _dispatch_body,
        grid=(1,),
        in_specs=[
            pl.BlockSpec((N, Dd), lambda i: (0, 0)),
            pl.BlockSpec((E, Dd), lambda i: (0, 0)),
            pl.BlockSpec((1, E), lambda i: (0, 0)),
        ],
        out_specs=[
            pl.BlockSpec((N, 1), lambda i: (0, 0)),
            pl.BlockSpec((E, 1), lambda i: (0, 0)),
            pl.BlockSpec((E, 1), lambda i: (0, 0)),
            pl.BlockSpec((E, 1), lambda i: (0, 0)),
            pl.BlockSpec((E, 1), lambda i: (0, 0)),
        ],
        out_shape=[
            jax.ShapeDtypeStruct((N, 1), i32),
            jax.ShapeDtypeStruct((E, 1), i32),
            jax.ShapeDtypeStruct((E, 1), i32),
            jax.ShapeDtypeStruct((E, 1), i32),
            jax.ShapeDtypeStruct((E, 1), i32),
        ],
    )(x_flat, gate_W, gate_b.reshape(1, E))
    posr = pos2.reshape(1, N)
    ts_a, te1_a, lo_a, hi_a = (
        a.reshape(E) for a in (ts_a, te1_a, lo_a, hi_a))

    x_sorted = _sc_scatter_rows(x_flat, posr)

    Fh = F // 2
    f32 = jnp.float32
    grid_spec = pltpu.PrefetchScalarGridSpec(
        num_scalar_prefetch=4,
        grid=(1,),
        in_specs=[
            pl.BlockSpec((N, Dd), lambda i, *s: (0, 0)),
            pl.BlockSpec(memory_space=pltpu.MemorySpace.HBM),
            pl.BlockSpec((E, 1, F), lambda i, *s: (0, 0, 0)),
            pl.BlockSpec(memory_space=pltpu.MemorySpace.HBM),
            pl.BlockSpec((E, 1, Dd), lambda i, *s: (0, 0, 0)),
        ],
        out_specs=pl.BlockSpec((N, Dd), lambda i, *s: (0, 0)),
        scratch_shapes=[
            pltpu.VMEM((_DEPTH, Fh, Dd), f32),
            pltpu.VMEM((_DEPTH, Dd, Fh), f32),
            pltpu.SemaphoreType.DMA((_DEPTH,)),
            pltpu.SemaphoreType.DMA((_DEPTH,)),
        ],
    )
    out_sorted = pl.pallas_call(
        _ffn_body,
        grid_spec=grid_spec,
        out_shape=jax.ShapeDtypeStruct((N, Dd), jnp.float32),
        compiler_params=pltpu.CompilerParams(vmem_limit_bytes=63 << 20),
    )(ts_a, te1_a, lo_a, hi_a,
      x_sorted, W1, b1.reshape(E, 1, F), W2, b2.reshape(E, 1, Dd))

    out = _sc_gather_rows(out_sorted, posr)
    return out.reshape(Bd, L, Dd)
